# baseline (speedup 1.0000x reference)
import jax
import jax.numpy as jnp
from jax import lax
from jax.experimental import pallas as pl
from jax.experimental.pallas import tpu as pltpu
from jax.experimental.pallas import tpu_sc as plsc

_NUM_CORES = 2
_NUM_SUBCORES = 16
_NW = _NUM_CORES * _NUM_SUBCORES

_B = 4096
_L = 50
_D = 32
_ROWS = _B * _L
_RPW = _ROWS // _NW
_BPW = _B // _NW
_SLEN = 128
_NSTREAM = _RPW // _SLEN
_SPC = 10
_NCHUNK = _NSTREAM // _SPC
_CROWS = _SPC * _SLEN
_MROWS = _ROWS // _SLEN


def _embed_body(table, q1d, lens, out, idx_v, mask_v, lens_v, buf0, buf1,
                g0, g1, o0, o1):
  wid = lax.axis_index("s") * _NUM_CORES + lax.axis_index("c")
  row0 = wid * _RPW

  pltpu.sync_copy(q1d.at[pl.ds(row0, _RPW)], idx_v)

  bufs = (buf0, buf1)
  gsem = (g0, g1)
  osem = (o0, o1)

  def fire_gather(g):
    buf = bufs[g % 2]
    return [
        pltpu.async_copy(table.at[idx_v.at[pl.ds((g * _SPC + s) * _SLEN, _SLEN)]],
                         buf.at[pl.ds(s * _SLEN, _SLEN)], gsem[g % 2])
        for s in range(_SPC)
    ]

  gh = [None] * _NCHUNK
  oh = [None] * _NCHUNK
  gh[0] = fire_gather(0)

  pltpu.sync_copy(lens.at[pl.ds(wid * _BPW, _BPW)], lens_v)

  def mask_body(j, _):
    p = j * 16 + lax.iota(jnp.int32, 16)
    r = lax.shift_right_logical(p * 5243, 18)
    c = p - r * _L
    lv = plsc.load_gather(lens_v, [r])
    mask_v[pl.ds(j * 16, 16)] = jnp.where(c < lv, jnp.float32(1.0),
                                          jnp.float32(0.0))
    return 0

  lax.fori_loop(0, _RPW // 16, mask_body, 0, unroll=4)

  for g in range(_NCHUNK):
    b = g % 2
    buf = bufs[b]
    if g + 1 < _NCHUNK:
      if g >= 1:
        oh[g - 1].wait()
      gh[g + 1] = fire_gather(g + 1)
    for h in gh[g]:
      h.wait()
    base = g * _CROWS

    def mblock(jb, _):
      mv = mask_v[pl.ds(base + jb * 16, 16)]
      rb = jb * 16
      for r in range(16):
        m = mv[r]
        buf[rb + r, pl.ds(0, 16)] = buf[rb + r, pl.ds(0, 16)] * m
        buf[rb + r, pl.ds(16, 16)] = buf[rb + r, pl.ds(16, 16)] * m
      return 0

    lax.fori_loop(0, _CROWS // 16, mblock, 0)
    oh[g] = pltpu.async_copy(buf, out.at[pl.ds(row0 + g * _CROWS, _CROWS)],
                             osem[b])
  oh[_NCHUNK - 2].wait()
  oh[_NCHUNK - 1].wait()


_V = 1000000
_CW = 768
_NCH = (-(-31256 // _CW) + 1) // 2 * 2


def _repack_body(tt, t128, in0, in1, ob0, ob1, si0, si1, so0, so1):
  wid = lax.axis_index("s") * _NUM_CORES + lax.axis_index("c")
  start = wid * (_V // _NW) // 8 * 8
  end = jnp.where(wid == _NW - 1, _V, (wid + 1) * (_V // _NW) // 8 * 8)

  ins = (in0, in1)
  obs = (ob0, ob1)
  isem = (si0, si1)
  osem = (so0, so1)

  def cstart(k):
    return jnp.minimum(start + k * _CW, end - _CW)

  def fire_in(k, b):
    pltpu.async_copy(tt.at[:, pl.ds(cstart(k), _CW)], ins[b], isem[b])

  fire_in(0, 0)
  fire_in(1, 1)

  r0 = lax.iota(jnp.int32, 16)
  r1 = r0 + 16

  def pair(j, _):
    for h in (0, 1):
      k = 2 * j + h
      inb = ins[h]
      ob = obs[h]
      pltpu.make_async_copy(tt.at[:, pl.ds(0, _CW)], inb, isem[h]).wait()

      @pl.when(k >= 2)
      def _():
        pltpu.make_async_copy(ob, t128.at[pl.ds(0, _CW * 32)],
                              osem[h]).wait()

      def trow(w, _, inb=inb, ob=ob):
        cw = jnp.full((16,), w, jnp.int32)
        ob[pl.ds(w * 32, 16)] = plsc.load_gather(inb, [r0, cw])
        ob[pl.ds(w * 32 + 16, 16)] = plsc.load_gather(inb, [r1, cw])
        return 0

      lax.fori_loop(0, _CW, trow, 0, unroll=8)
      pltpu.async_copy(ob, t128.at[pl.ds(cstart(k) * 32, _CW * 32)], osem[h])

      @pl.when(k + 2 < _NCH)
      def _():
        fire_in(k + 2, h)
    return 0

  lax.fori_loop(0, _NCH // 2, pair, 0)
  pltpu.make_async_copy(ob0, t128.at[pl.ds(0, _CW * 32)], osem[0]).wait()
  pltpu.make_async_copy(ob1, t128.at[pl.ds(0, _CW * 32)], osem[1]).wait()


def _sc_repack(table_t):
  mesh = plsc.VectorSubcoreMesh(core_axis_name="c", subcore_axis_name="s",
                                num_cores=_NUM_CORES,
                                num_subcores=_NUM_SUBCORES)
  return pl.kernel(
      _repack_body,
      out_type=jax.ShapeDtypeStruct((_V * _D,), jnp.float32),
      mesh=mesh,
      compiler_params=pltpu.CompilerParams(use_tc_tiling_on_sc=False,
                                           needs_layout_passes=False),
      scratch_types=[
          pltpu.VMEM((32, _CW), jnp.float32),
          pltpu.VMEM((32, _CW), jnp.float32),
          pltpu.VMEM((_CW * 32,), jnp.float32),
          pltpu.VMEM((_CW * 32,), jnp.float32),
          pltpu.SemaphoreType.DMA,
          pltpu.SemaphoreType.DMA,
          pltpu.SemaphoreType.DMA,
          pltpu.SemaphoreType.DMA,
      ],
  )(table_t)


def kernel(queries, query_lens, embedding_weight):
  q1d = queries.astype(jnp.int32).reshape(_ROWS)
  lens = query_lens.astype(jnp.int32)
  t128 = _sc_repack(embedding_weight.T)
  table_rm = t128.reshape(embedding_weight.shape)
  mesh = plsc.VectorSubcoreMesh(core_axis_name="c", subcore_axis_name="s",
                                num_cores=_NUM_CORES,
                                num_subcores=_NUM_SUBCORES)
  out = pl.kernel(
      _embed_body,
      out_type=jax.ShapeDtypeStruct((_ROWS, _D), jnp.float32),
      mesh=mesh,
      compiler_params=pltpu.CompilerParams(use_tc_tiling_on_sc=False,
                                           needs_layout_passes=False),
      scratch_types=[
          pltpu.VMEM((_RPW,), jnp.int32),
          pltpu.VMEM((_RPW,), jnp.float32),
          pltpu.VMEM((_BPW,), jnp.int32),
          pltpu.VMEM((_CROWS, _D), jnp.float32),
          pltpu.VMEM((_CROWS, _D), jnp.float32),
          pltpu.SemaphoreType.DMA,
          pltpu.SemaphoreType.DMA,
          pltpu.SemaphoreType.DMA,
          pltpu.SemaphoreType.DMA,
      ],
  )(table_rm, q1d, lens)
  return out.reshape(_B, _L, _D)


# speedup vs baseline: 2.4400x; 2.4400x over previous
import jax
import jax.numpy as jnp
from jax import lax
from jax.experimental import pallas as pl
from jax.experimental.pallas import tpu as pltpu
from jax.experimental.pallas import tpu_sc as plsc

_NUM_CORES = 2
_NUM_SUBCORES = 16
_NW = _NUM_CORES * _NUM_SUBCORES

_B = 4096
_L = 50
_D = 32
_ROWS = _B * _L
_RPW = _ROWS // _NW
_BPW = _B // _NW
_SLEN = 128
_NSTREAM = _RPW // _SLEN
_SPC = 10
_NCHUNK = _NSTREAM // _SPC
_CROWS = _SPC * _SLEN
_MROWS = _ROWS // _SLEN


def _embed_body(table, q1d, lens, out, idx_v, mask_v, lens_v, buf0, buf1,
                g0, g1, o0, o1):
  wid = lax.axis_index("s") * _NUM_CORES + lax.axis_index("c")
  row0 = wid * _RPW

  pltpu.sync_copy(q1d.at[pl.ds(row0, _RPW)], idx_v)

  bufs = (buf0, buf1)
  gsem = (g0, g1)
  osem = (o0, o1)

  def fire_gather(g):
    buf = bufs[g % 2]
    return [
        pltpu.async_copy(table.at[idx_v.at[pl.ds((g * _SPC + s) * _SLEN, _SLEN)]],
                         buf.at[pl.ds(s * _SLEN, _SLEN)], gsem[g % 2])
        for s in range(_SPC)
    ]

  gh = [None] * _NCHUNK
  oh = [None] * _NCHUNK
  gh[0] = fire_gather(0)

  pltpu.sync_copy(lens.at[pl.ds(wid * _BPW, _BPW)], lens_v)

  def mask_body(j, _):
    p = j * 16 + lax.iota(jnp.int32, 16)
    r = lax.shift_right_logical(p * 5243, 18)
    c = p - r * _L
    lv = plsc.load_gather(lens_v, [r])
    mask_v[pl.ds(j * 16, 16)] = jnp.where(c < lv, jnp.float32(1.0),
                                          jnp.float32(0.0))
    return 0

  lax.fori_loop(0, _RPW // 16, mask_body, 0, unroll=4)

  for g in range(_NCHUNK):
    b = g % 2
    buf = bufs[b]
    if g + 1 < _NCHUNK:
      if g >= 1:
        oh[g - 1].wait()
      gh[g + 1] = fire_gather(g + 1)
    for h in gh[g]:
      h.wait()
    base = g * _CROWS

    def mblock(jb, _):
      mv = mask_v[pl.ds(base + jb * 16, 16)]
      rb = jb * 16
      for r in range(16):
        m = mv[r]
        buf[rb + r, pl.ds(0, 16)] = buf[rb + r, pl.ds(0, 16)] * m
        buf[rb + r, pl.ds(16, 16)] = buf[rb + r, pl.ds(16, 16)] * m
      return 0

    lax.fori_loop(0, _CROWS // 16, mblock, 0)
    oh[g] = pltpu.async_copy(buf, out.at[pl.ds(row0 + g * _CROWS, _CROWS)],
                             osem[b])
  oh[_NCHUNK - 2].wait()
  oh[_NCHUNK - 1].wait()


_V = 1000000
_VPAD = 1000064
_DW = 4096
_NB = -(-_V // _DW)
_D1 = _NB * (_DW * 8 // 128)
_PLANE = _D1 * 128


def _dump_body(t_ref, o_ref):
  x = t_ref[...]
  x = x.reshape(8, _DW // 128, 128)
  x = jnp.transpose(x, (1, 0, 2))
  o_ref[...] = x.reshape(1, _DW * 8 // 128, 128)


def _tc_dump(table_t):
  return pl.pallas_call(
      _dump_body,
      grid=(4, _NB),
      in_specs=[pl.BlockSpec((8, _DW), lambda si, wb: (si, wb))],
      out_specs=pl.BlockSpec((1, _DW * 8 // 128, 128),
                             lambda si, wb: (si, wb, 0)),
      out_shape=jax.ShapeDtypeStruct((4, _D1, 128), jnp.float32),
  )(table_t)


_CW = 768
_CPS = _CW * 8
_NCH = 42


def _repack_body(td, t128, in0, in1, ob0, ob1, si0, si1, so0, so1):
  wid = lax.axis_index("s") * _NUM_CORES + lax.axis_index("c")
  start = wid * (_V // _NW) // 128 * 128
  end = jnp.where(wid == _NW - 1, _VPAD,
                  (wid + 1) * (_V // _NW) // 128 * 128)

  ins = (in0, in1)
  obs = (ob0, ob1)
  isem = (si0, si1)
  osem = (so0, so1)

  def cstart(k):
    return jnp.minimum(start + k * _CW, end - _CW)

  def fire_in(k, b):
    cs = cstart(k)
    for si in range(4):
      pltpu.async_copy(td.at[pl.ds(si * _PLANE + cs * 8, _CPS)],
                       ins[b].at[pl.ds(si * _CPS, _CPS)], isem[b])

  fire_in(0, 0)
  fire_in(1, 1)

  bscat = lax.iota(jnp.int32, 16) * 32

  def pair(j, _):
    for h in (0, 1):
      k = 2 * j + h
      inb = ins[h]
      ob = obs[h]
      for si in range(4):
        pltpu.make_async_copy(td.at[pl.ds(0, _CPS)],
                              inb.at[pl.ds(si * _CPS, _CPS)], isem[h]).wait()

      @pl.when(k >= 2)
      def _():
        pltpu.make_async_copy(ob, t128.at[pl.ds(0, _CW * 32)],
                              osem[h]).wait()

      def grp(g, _, inb=inb, ob=ob):
        x = g * 16
        woff = lax.shift_right_logical(x, 7) * 1024 + (x & 127)
        for c in range(32):
          si, s = c // 8, c % 8
          v = inb[pl.ds(si * _CPS + s * 128 + woff, 16)]
          plsc.store_scatter(ob, [bscat + (x * 32 + c)], v)
        return 0

      lax.fori_loop(0, _CW // 16, grp, 0)
      pltpu.async_copy(ob, t128.at[pl.ds(cstart(k) * 32, _CW * 32)], osem[h])

      @pl.when(k + 2 < _NCH)
      def _():
        fire_in(k + 2, h)
    return 0

  lax.fori_loop(0, _NCH // 2, pair, 0)
  pltpu.make_async_copy(ob0, t128.at[pl.ds(0, _CW * 32)], osem[0]).wait()
  pltpu.make_async_copy(ob1, t128.at[pl.ds(0, _CW * 32)], osem[1]).wait()


def _sc_repack(td_flat):
  mesh = plsc.VectorSubcoreMesh(core_axis_name="c", subcore_axis_name="s",
                                num_cores=_NUM_CORES,
                                num_subcores=_NUM_SUBCORES)
  return pl.kernel(
      _repack_body,
      out_type=jax.ShapeDtypeStruct((_VPAD * _D,), jnp.float32),
      mesh=mesh,
      compiler_params=pltpu.CompilerParams(use_tc_tiling_on_sc=False,
                                           needs_layout_passes=False),
      scratch_types=[
          pltpu.VMEM((4 * _CPS,), jnp.float32),
          pltpu.VMEM((4 * _CPS,), jnp.float32),
          pltpu.VMEM((_CW * 32,), jnp.float32),
          pltpu.VMEM((_CW * 32,), jnp.float32),
          pltpu.SemaphoreType.DMA,
          pltpu.SemaphoreType.DMA,
          pltpu.SemaphoreType.DMA,
          pltpu.SemaphoreType.DMA,
      ],
  )(td_flat)


def kernel(queries, query_lens, embedding_weight):
  q1d = queries.astype(jnp.int32).reshape(_ROWS)
  lens = query_lens.astype(jnp.int32)
  td = _tc_dump(embedding_weight.T)
  t128 = _sc_repack(td.reshape(-1))
  table_rm = t128.reshape(_VPAD, _D)
  mesh = plsc.VectorSubcoreMesh(core_axis_name="c", subcore_axis_name="s",
                                num_cores=_NUM_CORES,
                                num_subcores=_NUM_SUBCORES)
  out = pl.kernel(
      _embed_body,
      out_type=jax.ShapeDtypeStruct((_ROWS, _D), jnp.float32),
      mesh=mesh,
      compiler_params=pltpu.CompilerParams(use_tc_tiling_on_sc=False,
                                           needs_layout_passes=False),
      scratch_types=[
          pltpu.VMEM((_RPW,), jnp.int32),
          pltpu.VMEM((_RPW,), jnp.float32),
          pltpu.VMEM((_BPW,), jnp.int32),
          pltpu.VMEM((_CROWS, _D), jnp.float32),
          pltpu.VMEM((_CROWS, _D), jnp.float32),
          pltpu.SemaphoreType.DMA,
          pltpu.SemaphoreType.DMA,
          pltpu.SemaphoreType.DMA,
          pltpu.SemaphoreType.DMA,
      ],
  )(table_rm, q1d, lens)
  return out.reshape(_B, _L, _D)


# speedup vs baseline: 4.4431x; 1.8209x over previous
import jax
import jax.numpy as jnp
from jax import lax
from jax.experimental import pallas as pl
from jax.experimental.pallas import tpu as pltpu
from jax.experimental.pallas import tpu_sc as plsc

_NUM_CORES = 2
_NUM_SUBCORES = 16
_NW = _NUM_CORES * _NUM_SUBCORES

_B = 4096
_L = 50
_D = 32
_ROWS = _B * _L
_RPW = _ROWS // _NW
_BPW = _B // _NW
_SLEN = 128
_NSTREAM = _RPW // _SLEN
_SPC = 10
_NCHUNK = _NSTREAM // _SPC
_CROWS = _SPC * _SLEN
_MROWS = _ROWS // _SLEN


def _embed_body(table, q1d, lens, out, idx_v, mask_v, lens_v, buf0, buf1,
                g0, g1, o0, o1):
  wid = lax.axis_index("s") * _NUM_CORES + lax.axis_index("c")
  row0 = wid * _RPW

  pltpu.sync_copy(q1d.at[pl.ds(row0, _RPW)], idx_v)

  bufs = (buf0, buf1)
  gsem = (g0, g1)
  osem = (o0, o1)

  def fire_gather(g):
    buf = bufs[g % 2]
    return [
        pltpu.async_copy(table.at[idx_v.at[pl.ds((g * _SPC + s) * _SLEN, _SLEN)]],
                         buf.at[pl.ds(s * _SLEN, _SLEN)], gsem[g % 2])
        for s in range(_SPC)
    ]

  gh = [None] * _NCHUNK
  oh = [None] * _NCHUNK
  gh[0] = fire_gather(0)

  pltpu.sync_copy(lens.at[pl.ds(wid * _BPW, _BPW)], lens_v)

  def mask_body(j, _):
    p = j * 16 + lax.iota(jnp.int32, 16)
    r = lax.shift_right_logical(p * 5243, 18)
    c = p - r * _L
    lv = plsc.load_gather(lens_v, [r])
    mask_v[pl.ds(j * 16, 16)] = jnp.where(c < lv, jnp.float32(1.0),
                                          jnp.float32(0.0))
    return 0

  lax.fori_loop(0, _RPW // 16, mask_body, 0, unroll=4)

  for g in range(_NCHUNK):
    b = g % 2
    buf = bufs[b]
    if g + 1 < _NCHUNK:
      if g >= 1:
        oh[g - 1].wait()
      gh[g + 1] = fire_gather(g + 1)
    for h in gh[g]:
      h.wait()
    base = g * _CROWS

    def mblock(jb, _):
      mv = mask_v[pl.ds(base + jb * 16, 16)]
      rb = jb * 16
      for r in range(16):
        m = mv[r]
        buf[rb + r, pl.ds(0, 16)] = buf[rb + r, pl.ds(0, 16)] * m
        buf[rb + r, pl.ds(16, 16)] = buf[rb + r, pl.ds(16, 16)] * m
      return 0

    lax.fori_loop(0, _CROWS // 16, mblock, 0)
    oh[g] = pltpu.async_copy(buf, out.at[pl.ds(row0 + g * _CROWS, _CROWS)],
                             osem[b])
  oh[_NCHUNK - 2].wait()
  oh[_NCHUNK - 1].wait()


_V = 1000000
_VPAD = 1000064
_DW = 32768
_NB = -(-_V // _DW)
_D1 = _NB * (_DW * 8 // 128)
_PLANE = _D1 * 128


def _dump_body(t_ref, o_ref):
  x = t_ref[...]
  x = x.reshape(8, _DW // 128, 128)
  x = jnp.transpose(x, (1, 0, 2))
  o_ref[...] = x.reshape(1, _DW * 8 // 128, 128)


def _tc_dump(table_t):
  return pl.pallas_call(
      _dump_body,
      grid=(4, _NB),
      in_specs=[pl.BlockSpec((8, _DW), lambda si, wb: (si, wb))],
      out_specs=pl.BlockSpec((1, _DW * 8 // 128, 128),
                             lambda si, wb: (si, wb, 0)),
      out_shape=jax.ShapeDtypeStruct((4, _D1, 128), jnp.float32),
  )(table_t)


_CW = 768
_CR = _CW // 128 * 8
_IP = 129
_NCH = 42


def _repack_body(td, t128, in0, in1, ob0, ob1, si0, si1, so0, so1):
  wid = lax.axis_index("s") * _NUM_CORES + lax.axis_index("c")
  start = wid * (_V // _NW) // 128 * 128
  end = jnp.where(wid == _NW - 1, _VPAD,
                  (wid + 1) * (_V // _NW) // 128 * 128)

  ins = (in0, in1)
  obs = (ob0, ob1)
  isem = (si0, si1)
  osem = (so0, so1)

  def cstart(k):
    return jnp.minimum(start + k * _CW, end - _CW)

  def fire_in(k, b):
    rs = cstart(k) // 128 * 8
    for si in range(4):
      pltpu.async_copy(td.at[si, pl.ds(rs, _CR), :],
                       ins[b].at[pl.ds(si * _CR, _CR), pl.ds(0, 128)],
                       isem[b])

  fire_in(0, 0)
  fire_in(1, 1)

  cj = lax.iota(jnp.int32, 16)
  crow0 = (cj // 8) * _CR + cj % 8
  crow1 = ((cj + 16) // 8) * _CR + cj % 8

  def pair(j, _):
    for h in (0, 1):
      k = 2 * j + h
      inb = ins[h]
      ob = obs[h]
      for si in range(4):
        pltpu.make_async_copy(td.at[0, pl.ds(0, _CR), :],
                              inb.at[pl.ds(si * _CR, _CR), pl.ds(0, 128)],
                              isem[h]).wait()

      @pl.when(k >= 2)
      def _():
        pltpu.make_async_copy(ob, t128.at[pl.ds(0, _CW * 32)],
                              osem[h]).wait()

      def grp(w, _, inb=inb, ob=ob):
        wj8 = lax.shift_right_logical(w, 7) * 8
        u = jnp.full((16,), w & 127, jnp.int32)
        ob[pl.ds(w * 32, 16)] = plsc.load_gather(inb, [crow0 + wj8, u])
        ob[pl.ds(w * 32 + 16, 16)] = plsc.load_gather(inb, [crow1 + wj8, u])
        return 0

      lax.fori_loop(0, _CW, grp, 0, unroll=4)
      pltpu.async_copy(ob, t128.at[pl.ds(cstart(k) * 32, _CW * 32)], osem[h])

      @pl.when(k + 2 < _NCH)
      def _():
        fire_in(k + 2, h)
    return 0

  lax.fori_loop(0, _NCH // 2, pair, 0)
  pltpu.make_async_copy(ob0, t128.at[pl.ds(0, _CW * 32)], osem[0]).wait()
  pltpu.make_async_copy(ob1, t128.at[pl.ds(0, _CW * 32)], osem[1]).wait()


def _sc_repack(td):
  mesh = plsc.VectorSubcoreMesh(core_axis_name="c", subcore_axis_name="s",
                                num_cores=_NUM_CORES,
                                num_subcores=_NUM_SUBCORES)
  return pl.kernel(
      _repack_body,
      out_type=jax.ShapeDtypeStruct((_VPAD * _D,), jnp.float32),
      mesh=mesh,
      compiler_params=pltpu.CompilerParams(use_tc_tiling_on_sc=False,
                                           needs_layout_passes=False),
      scratch_types=[
          pltpu.VMEM((4 * _CR, _IP), jnp.float32),
          pltpu.VMEM((4 * _CR, _IP), jnp.float32),
          pltpu.VMEM((_CW * 32,), jnp.float32),
          pltpu.VMEM((_CW * 32,), jnp.float32),
          pltpu.SemaphoreType.DMA,
          pltpu.SemaphoreType.DMA,
          pltpu.SemaphoreType.DMA,
          pltpu.SemaphoreType.DMA,
      ],
  )(td)


def kernel(queries, query_lens, embedding_weight):
  q1d = queries.astype(jnp.int32).reshape(_ROWS)
  lens = query_lens.astype(jnp.int32)
  td = _tc_dump(embedding_weight.T)
  t128 = _sc_repack(td)
  table_rm = t128.reshape(_VPAD, _D)
  mesh = plsc.VectorSubcoreMesh(core_axis_name="c", subcore_axis_name="s",
                                num_cores=_NUM_CORES,
                                num_subcores=_NUM_SUBCORES)
  out = pl.kernel(
      _embed_body,
      out_type=jax.ShapeDtypeStruct((_ROWS, _D), jnp.float32),
      mesh=mesh,
      compiler_params=pltpu.CompilerParams(use_tc_tiling_on_sc=False,
                                           needs_layout_passes=False),
      scratch_types=[
          pltpu.VMEM((_RPW,), jnp.int32),
          pltpu.VMEM((_RPW,), jnp.float32),
          pltpu.VMEM((_BPW,), jnp.int32),
          pltpu.VMEM((_CROWS, _D), jnp.float32),
          pltpu.VMEM((_CROWS, _D), jnp.float32),
          pltpu.SemaphoreType.DMA,
          pltpu.SemaphoreType.DMA,
          pltpu.SemaphoreType.DMA,
          pltpu.SemaphoreType.DMA,
      ],
  )(table_rm, q1d, lens)
  return out.reshape(_B, _L, _D)


# speedup vs baseline: 6.5394x; 1.4718x over previous
import jax
import jax.numpy as jnp
from jax import lax
from jax.experimental import pallas as pl
from jax.experimental.pallas import tpu as pltpu
from jax.experimental.pallas import tpu_sc as plsc

_NUM_CORES = 2
_NUM_SUBCORES = 16
_NW = _NUM_CORES * _NUM_SUBCORES

_B = 4096
_L = 50
_D = 32
_ROWS = _B * _L
_RPW = _ROWS // _NW
_BPW = _B // _NW
_SLEN = 128
_NSTREAM = _RPW // _SLEN
_SPC = 10
_NCHUNK = _NSTREAM // _SPC
_CROWS = _SPC * _SLEN
_MROWS = _ROWS // _SLEN


def _embed_body(table, q1d, lens, out, idx_v, mask_v, lens_v, buf0, buf1,
                g0, g1, o0, o1):
  wid = lax.axis_index("s") * _NUM_CORES + lax.axis_index("c")
  row0 = wid * _RPW

  pltpu.sync_copy(q1d.at[pl.ds(row0, _RPW)], idx_v)

  bufs = (buf0, buf1)
  gsem = (g0, g1)
  osem = (o0, o1)

  def fire_gather(g):
    buf = bufs[g % 2]
    return [
        pltpu.async_copy(table.at[idx_v.at[pl.ds((g * _SPC + s) * _SLEN, _SLEN)]],
                         buf.at[pl.ds(s * _SLEN, _SLEN)], gsem[g % 2])
        for s in range(_SPC)
    ]

  gh = [None] * _NCHUNK
  oh = [None] * _NCHUNK
  gh[0] = fire_gather(0)

  pltpu.sync_copy(lens.at[pl.ds(wid * _BPW, _BPW)], lens_v)

  def mask_body(j, _):
    p = j * 16 + lax.iota(jnp.int32, 16)
    r = lax.shift_right_logical(p * 5243, 18)
    c = p - r * _L
    lv = plsc.load_gather(lens_v, [r])
    mask_v[pl.ds(j * 16, 16)] = jnp.where(c < lv, jnp.float32(1.0),
                                          jnp.float32(0.0))
    return 0

  lax.fori_loop(0, _RPW // 16, mask_body, 0, unroll=4)

  for g in range(_NCHUNK):
    b = g % 2
    buf = bufs[b]
    if g + 1 < _NCHUNK:
      if g >= 1:
        oh[g - 1].wait()
      gh[g + 1] = fire_gather(g + 1)
    for h in gh[g]:
      h.wait()
    base = g * _CROWS

    def mblock(jb, _):
      mv = mask_v[pl.ds(base + jb * 16, 16)]
      rb = jb * 16
      for r in range(16):
        m = mv[r]
        buf[rb + r, pl.ds(0, 16)] = buf[rb + r, pl.ds(0, 16)] * m
        buf[rb + r, pl.ds(16, 16)] = buf[rb + r, pl.ds(16, 16)] * m
      return 0

    lax.fori_loop(0, _CROWS // 16, mblock, 0)
    oh[g] = pltpu.async_copy(buf, out.at[pl.ds(row0 + g * _CROWS, _CROWS)],
                             osem[b])
  oh[_NCHUNK - 2].wait()
  oh[_NCHUNK - 1].wait()


_V = 1000000
_VPAD = 1000064
_DW = 32768
_NB = -(-_V // _DW)
_D1 = _NB * (_DW * 8 // 128)
_PLANE = _D1 * 128


def _dump_body(t_ref, o_ref):
  x = t_ref[...]
  x = x.reshape(8, _DW // 128, 128)
  x = jnp.transpose(x, (1, 0, 2))
  o_ref[...] = x.reshape(1, _DW * 8 // 128, 128)


def _tc_dump(table_t):
  return pl.pallas_call(
      _dump_body,
      grid=(4, _NB),
      in_specs=[pl.BlockSpec((8, _DW), lambda si, wb: (si, wb))],
      out_specs=pl.BlockSpec((1, _DW * 8 // 128, 128),
                             lambda si, wb: (si, wb, 0)),
      out_shape=jax.ShapeDtypeStruct((4, _D1, 128), jnp.float32),
  )(table_t)


_CW = 768
_CR = _CW // 128 * 8
_IP = 129
_NCH = 42


def _repack_body(td, t128, in0, in1, ob0, ob1, si0, si1, so0, so1):
  wid = lax.axis_index("s") * _NUM_CORES + lax.axis_index("c")
  start = wid * (_V // _NW) // 128 * 128
  end = jnp.where(wid == _NW - 1, _VPAD,
                  (wid + 1) * (_V // _NW) // 128 * 128)

  ins = (in0, in1)
  obs = (ob0, ob1)
  isem = (si0, si1)
  osem = (so0, so1)

  def cstart(k):
    return jnp.minimum(start + k * _CW, end - _CW)

  def fire_in(k, b):
    rs = cstart(k) // 128 * 8
    for si in range(4):
      pltpu.async_copy(td.at[si, pl.ds(rs, _CR), :],
                       ins[b].at[pl.ds(si * _CR, _CR), pl.ds(0, 128)],
                       isem[b])

  fire_in(0, 0)
  fire_in(1, 1)

  cj = lax.iota(jnp.int32, 16)
  crow0 = (cj // 8) * _CR + cj % 8
  crow1 = ((cj + 16) // 8) * _CR + cj % 8

  def pair(j, _):
    for h in (0, 1):
      k = 2 * j + h
      inb = ins[h]
      ob = obs[h]
      for si in range(4):
        pltpu.make_async_copy(td.at[0, pl.ds(0, _CR), :],
                              inb.at[pl.ds(si * _CR, _CR), pl.ds(0, 128)],
                              isem[h]).wait()

      @pl.when(k >= 2)
      def _():
        pltpu.make_async_copy(ob, t128.at[pl.ds(0, _CW * 32)],
                              osem[h]).wait()

      @plsc.parallel_loop(0, _CW, unroll=8)
      def _(w, inb=inb, ob=ob):
        wj8 = lax.shift_right_logical(w, 7) * 8
        u = jnp.full((16,), w & 127, jnp.int32)
        ob[pl.ds(w * 32, 16)] = plsc.load_gather(inb, [crow0 + wj8, u])
        ob[pl.ds(w * 32 + 16, 16)] = plsc.load_gather(inb, [crow1 + wj8, u])
      pltpu.async_copy(ob, t128.at[pl.ds(cstart(k) * 32, _CW * 32)], osem[h])

      @pl.when(k + 2 < _NCH)
      def _():
        fire_in(k + 2, h)
    return 0

  lax.fori_loop(0, _NCH // 2, pair, 0)
  pltpu.make_async_copy(ob0, t128.at[pl.ds(0, _CW * 32)], osem[0]).wait()
  pltpu.make_async_copy(ob1, t128.at[pl.ds(0, _CW * 32)], osem[1]).wait()


def _sc_repack(td):
  mesh = plsc.VectorSubcoreMesh(core_axis_name="c", subcore_axis_name="s",
                                num_cores=_NUM_CORES,
                                num_subcores=_NUM_SUBCORES)
  return pl.kernel(
      _repack_body,
      out_type=jax.ShapeDtypeStruct((_VPAD * _D,), jnp.float32),
      mesh=mesh,
      compiler_params=pltpu.CompilerParams(use_tc_tiling_on_sc=False,
                                           needs_layout_passes=False),
      scratch_types=[
          pltpu.VMEM((4 * _CR, _IP), jnp.float32),
          pltpu.VMEM((4 * _CR, _IP), jnp.float32),
          pltpu.VMEM((_CW * 32,), jnp.float32),
          pltpu.VMEM((_CW * 32,), jnp.float32),
          pltpu.SemaphoreType.DMA,
          pltpu.SemaphoreType.DMA,
          pltpu.SemaphoreType.DMA,
          pltpu.SemaphoreType.DMA,
      ],
  )(td)


def kernel(queries, query_lens, embedding_weight):
  q1d = queries.astype(jnp.int32).reshape(_ROWS)
  lens = query_lens.astype(jnp.int32)
  td = _tc_dump(embedding_weight.T)
  t128 = _sc_repack(td)
  table_rm = t128.reshape(_VPAD, _D)
  mesh = plsc.VectorSubcoreMesh(core_axis_name="c", subcore_axis_name="s",
                                num_cores=_NUM_CORES,
                                num_subcores=_NUM_SUBCORES)
  out = pl.kernel(
      _embed_body,
      out_type=jax.ShapeDtypeStruct((_ROWS, _D), jnp.float32),
      mesh=mesh,
      compiler_params=pltpu.CompilerParams(use_tc_tiling_on_sc=False,
                                           needs_layout_passes=False),
      scratch_types=[
          pltpu.VMEM((_RPW,), jnp.int32),
          pltpu.VMEM((_RPW,), jnp.float32),
          pltpu.VMEM((_BPW,), jnp.int32),
          pltpu.VMEM((_CROWS, _D), jnp.float32),
          pltpu.VMEM((_CROWS, _D), jnp.float32),
          pltpu.SemaphoreType.DMA,
          pltpu.SemaphoreType.DMA,
          pltpu.SemaphoreType.DMA,
          pltpu.SemaphoreType.DMA,
      ],
  )(table_rm, q1d, lens)
  return out.reshape(_B, _L, _D)


# speedup vs baseline: 11.7236x; 1.7928x over previous
import jax
import jax.numpy as jnp
from jax import lax
from jax.experimental import pallas as pl
from jax.experimental.pallas import tpu as pltpu
from jax.experimental.pallas import tpu_sc as plsc

_NUM_CORES = 2
_NUM_SUBCORES = 16
_NW = _NUM_CORES * _NUM_SUBCORES

_B = 4096
_L = 50
_D = 32
_ROWS = _B * _L
_RPW = _ROWS // _NW
_BPW = _B // _NW
_SLEN = 128
_NSTREAM = _RPW // _SLEN
_SPC = 10
_NCHUNK = _NSTREAM // _SPC
_CROWS = _SPC * _SLEN
_MROWS = _ROWS // _SLEN


_LPC = 5
_NLC = _L // _LPC
_CR3 = _LPC * _BPW
_OPIT = 129


def _embed_body(table, q1d, lens, out, idx_v, lens_v, ic0, ic1, rb0, rb1,
                ob0, ob1, g0, g1, o0, o1):
  wid = lax.axis_index("s") * _NUM_CORES + lax.axis_index("c")
  row0 = wid * _RPW

  pltpu.sync_copy(q1d.at[pl.ds(row0, _RPW)], idx_v)
  pltpu.sync_copy(lens.at[pl.ds(wid * _BPW, _BPW)], lens_v)

  ics = (ic0, ic1)
  rbs = (rb0, rb1)
  obs = (ob0, ob1)
  gsem = (g0, g1)
  osem = (o0, o1)

  b50 = lax.iota(jnp.int32, 16) * 50
  cj = lax.iota(jnp.int32, 16)
  rv0 = (cj // 8) * 8 + cj % 8
  rv1 = rv0 + 16

  def build_idx(ch, h):
    ic = ics[h]

    @plsc.parallel_loop(0, _LPC * (_BPW // 16), unroll=4)
    def _(g, ic=ic):
      lp = g // (_BPW // 16)
      b0 = (g % (_BPW // 16)) * 16
      v = plsc.load_gather(idx_v, [b50 + (b0 * 50 + ch * _LPC + lp)])
      ic[pl.ds(lp * _BPW + b0, 16)] = v

  def fire_gather(h):
    ic = ics[h]
    rb = rbs[h]
    for s in range(_CR3 // _SLEN):
      pltpu.async_copy(table.at[ic.at[pl.ds(s * _SLEN, _SLEN)]],
                       rb.at[pl.ds(s * _SLEN, _SLEN), :], gsem[h])

  build_idx(0, 0)
  fire_gather(0)
  build_idx(1, 1)
  fire_gather(1)

  def pair(j, _):
    for h in (0, 1):
      ch = 2 * j + h
      rb = rbs[h]
      ob = obs[h]
      for s in range(_CR3 // _SLEN):
        pltpu.make_async_copy(table.at[pl.ds(0, _SLEN)],
                              rb.at[pl.ds(0, _SLEN), :], gsem[h]).wait()

      @pl.when(ch >= 2)
      def _():
        for p in range(_LPC * 4):
          pltpu.make_async_copy(ob.at[pl.ds(0, 8), pl.ds(0, 128)],
                                out.at[0, 0, 0], osem[h]).wait()

      l0 = ch * _LPC

      @plsc.parallel_loop(0, _LPC * (_BPW // 16), unroll=2)
      def _(g, rb=rb, ob=ob, l0=l0):
        lp = g // (_BPW // 16)
        b0 = (g % (_BPW // 16)) * 16
        lens16 = lens_v[pl.ds(b0, 16)]
        mv = jnp.where(l0 + lp < lens16, jnp.float32(1.0), jnp.float32(0.0))
        for jj in range(16):
          m = mv[jj]
          r = lp * _BPW + b0 + jj
          col = jnp.full((16,), b0 + jj, jnp.int32)
          plsc.store_scatter(ob, [rv0 + lp * 32, col],
                             rb[r, pl.ds(0, 16)] * m)
          plsc.store_scatter(ob, [rv1 + lp * 32, col],
                             rb[r, pl.ds(16, 16)] * m)

      for lp in range(_LPC):
        for cb in range(4):
          pltpu.async_copy(
              ob.at[pl.ds((lp * 4 + cb) * 8, 8), pl.ds(0, 128)],
              out.at[l0 + lp, cb, wid], osem[h])

      @pl.when(ch + 2 < _NLC)
      def _():
        build_idx(ch + 2, h)
        fire_gather(h)
    return 0

  lax.fori_loop(0, _NLC // 2, pair, 0)
  for h in (0, 1):
    for p in range(_LPC * 4):
      pltpu.make_async_copy(obs[h].at[pl.ds(0, 8), pl.ds(0, 128)],
                            out.at[0, 0, 0], osem[h]).wait()


_V = 1000000
_VPAD = 1000064
_DW = 32768
_NB = -(-_V // _DW)
_D1 = _NB * (_DW * 8 // 128)
_PLANE = _D1 * 128


def _dump_body(t_ref, o_ref):
  x = t_ref[...]
  x = x.reshape(8, _DW // 128, 128)
  x = jnp.transpose(x, (1, 0, 2))
  o_ref[...] = x.reshape(1, _DW * 8 // 128, 128)


def _tc_dump(table_t):
  return pl.pallas_call(
      _dump_body,
      grid=(4, _NB),
      in_specs=[pl.BlockSpec((8, _DW), lambda si, wb: (si, wb))],
      out_specs=pl.BlockSpec((1, _DW * 8 // 128, 128),
                             lambda si, wb: (si, wb, 0)),
      out_shape=jax.ShapeDtypeStruct((4, _D1, 128), jnp.float32),
  )(table_t)


_CW = 768
_CR = _CW // 128 * 8
_IP = 129
_NCH = 42


def _repack_body(td, t128, in0, in1, ob0, ob1, si0, si1, so0, so1):
  wid = lax.axis_index("s") * _NUM_CORES + lax.axis_index("c")
  start = wid * (_V // _NW) // 128 * 128
  end = jnp.where(wid == _NW - 1, _VPAD,
                  (wid + 1) * (_V // _NW) // 128 * 128)

  ins = (in0, in1)
  obs = (ob0, ob1)
  isem = (si0, si1)
  osem = (so0, so1)

  def cstart(k):
    return jnp.minimum(start + k * _CW, end - _CW)

  def fire_in(k, b):
    rs = cstart(k) // 128 * 8
    for si in range(4):
      pltpu.async_copy(td.at[si, pl.ds(rs, _CR), :],
                       ins[b].at[pl.ds(si * _CR, _CR), pl.ds(0, 128)],
                       isem[b])

  fire_in(0, 0)
  fire_in(1, 1)

  cj = lax.iota(jnp.int32, 16)
  crow0 = (cj // 8) * _CR + cj % 8
  crow1 = ((cj + 16) // 8) * _CR + cj % 8

  def pair(j, _):
    for h in (0, 1):
      k = 2 * j + h
      inb = ins[h]
      ob = obs[h]
      for si in range(4):
        pltpu.make_async_copy(td.at[0, pl.ds(0, _CR), :],
                              inb.at[pl.ds(si * _CR, _CR), pl.ds(0, 128)],
                              isem[h]).wait()

      @pl.when(k >= 2)
      def _():
        pltpu.make_async_copy(ob, t128.at[pl.ds(0, _CW * 32)],
                              osem[h]).wait()

      @plsc.parallel_loop(0, _CW, unroll=8)
      def _(w, inb=inb, ob=ob):
        wj8 = lax.shift_right_logical(w, 7) * 8
        u = jnp.full((16,), w & 127, jnp.int32)
        ob[pl.ds(w * 32, 16)] = plsc.load_gather(inb, [crow0 + wj8, u])
        ob[pl.ds(w * 32 + 16, 16)] = plsc.load_gather(inb, [crow1 + wj8, u])
      pltpu.async_copy(ob, t128.at[pl.ds(cstart(k) * 32, _CW * 32)], osem[h])

      @pl.when(k + 2 < _NCH)
      def _():
        fire_in(k + 2, h)
    return 0

  lax.fori_loop(0, _NCH // 2, pair, 0)
  pltpu.make_async_copy(ob0, t128.at[pl.ds(0, _CW * 32)], osem[0]).wait()
  pltpu.make_async_copy(ob1, t128.at[pl.ds(0, _CW * 32)], osem[1]).wait()


def _sc_repack(td):
  mesh = plsc.VectorSubcoreMesh(core_axis_name="c", subcore_axis_name="s",
                                num_cores=_NUM_CORES,
                                num_subcores=_NUM_SUBCORES)
  return pl.kernel(
      _repack_body,
      out_type=jax.ShapeDtypeStruct((_VPAD * _D,), jnp.float32),
      mesh=mesh,
      compiler_params=pltpu.CompilerParams(use_tc_tiling_on_sc=False,
                                           needs_layout_passes=False),
      scratch_types=[
          pltpu.VMEM((4 * _CR, _IP), jnp.float32),
          pltpu.VMEM((4 * _CR, _IP), jnp.float32),
          pltpu.VMEM((_CW * 32,), jnp.float32),
          pltpu.VMEM((_CW * 32,), jnp.float32),
          pltpu.SemaphoreType.DMA,
          pltpu.SemaphoreType.DMA,
          pltpu.SemaphoreType.DMA,
          pltpu.SemaphoreType.DMA,
      ],
  )(td)


def kernel(queries, query_lens, embedding_weight):
  q1d = queries.astype(jnp.int32).reshape(_ROWS)
  lens = query_lens.astype(jnp.int32)
  td = _tc_dump(embedding_weight.T)
  t128 = _sc_repack(td)
  table_rm = t128.reshape(_VPAD, _D)
  mesh = plsc.VectorSubcoreMesh(core_axis_name="c", subcore_axis_name="s",
                                num_cores=_NUM_CORES,
                                num_subcores=_NUM_SUBCORES)
  out5 = pl.kernel(
      _embed_body,
      out_type=jax.ShapeDtypeStruct((_L, 4, _NW, 8, 128), jnp.float32),
      mesh=mesh,
      compiler_params=pltpu.CompilerParams(use_tc_tiling_on_sc=False,
                                           needs_layout_passes=False),
      scratch_types=[
          pltpu.VMEM((_RPW,), jnp.int32),
          pltpu.VMEM((_BPW,), jnp.int32),
          pltpu.VMEM((_CR3,), jnp.int32),
          pltpu.VMEM((_CR3,), jnp.int32),
          pltpu.VMEM((_CR3, _D), jnp.float32),
          pltpu.VMEM((_CR3, _D), jnp.float32),
          pltpu.VMEM((_LPC * 4 * 8, _OPIT), jnp.float32),
          pltpu.VMEM((_LPC * 4 * 8, _OPIT), jnp.float32),
          pltpu.SemaphoreType.DMA,
          pltpu.SemaphoreType.DMA,
          pltpu.SemaphoreType.DMA,
          pltpu.SemaphoreType.DMA,
      ],
  )(table_rm, q1d, lens)
  return out5.transpose(2, 4, 0, 1, 3).reshape(_B, _L, _D)


# speedup vs baseline: 11.7344x; 1.0009x over previous
import jax
import jax.numpy as jnp
from jax import lax
from jax.experimental import pallas as pl
from jax.experimental.pallas import tpu as pltpu
from jax.experimental.pallas import tpu_sc as plsc

_NUM_CORES = 2
_NUM_SUBCORES = 16
_NW = _NUM_CORES * _NUM_SUBCORES

_B = 4096
_L = 50
_D = 32
_ROWS = _B * _L
_RPW = _ROWS // _NW
_BPW = _B // _NW
_SLEN = 128


_LPC = 5
_NLC = _L // _LPC
_CR3 = _LPC * _BPW
_OPIT = 129


def _embed_body(table, q1d, lens, out, idx_v, lens_v, ic0, ic1, rb0, rb1,
                ob0, ob1, g0, g1, o0, o1):
  wid = lax.axis_index("s") * _NUM_CORES + lax.axis_index("c")
  row0 = wid * _RPW

  pltpu.sync_copy(q1d.at[pl.ds(row0, _RPW)], idx_v)
  pltpu.sync_copy(lens.at[pl.ds(wid * _BPW, _BPW)], lens_v)

  ics = (ic0, ic1)
  rbs = (rb0, rb1)
  obs = (ob0, ob1)
  gsem = (g0, g1)
  osem = (o0, o1)

  b50 = lax.iota(jnp.int32, 16) * 50
  cj = lax.iota(jnp.int32, 16)
  rv0 = (cj // 8) * 8 + cj % 8
  rv1 = rv0 + 16

  def build_idx(ch, h):
    ic = ics[h]

    @plsc.parallel_loop(0, _LPC * (_BPW // 16), unroll=4)
    def _(g, ic=ic):
      lp = g // (_BPW // 16)
      b0 = (g % (_BPW // 16)) * 16
      v = plsc.load_gather(idx_v, [b50 + (b0 * 50 + ch * _LPC + lp)])
      ic[pl.ds(lp * _BPW + b0, 16)] = v

  def fire_gather(h):
    ic = ics[h]
    rb = rbs[h]
    for s in range(_CR3 // _SLEN):
      pltpu.async_copy(table.at[ic.at[pl.ds(s * _SLEN, _SLEN)]],
                       rb.at[pl.ds(s * _SLEN, _SLEN), :], gsem[h])

  build_idx(0, 0)
  fire_gather(0)
  build_idx(1, 1)
  fire_gather(1)

  def pair(j, _):
    for h in (0, 1):
      ch = 2 * j + h
      rb = rbs[h]
      ob = obs[h]
      for s in range(_CR3 // _SLEN):
        pltpu.make_async_copy(table.at[pl.ds(0, _SLEN)],
                              rb.at[pl.ds(0, _SLEN), :], gsem[h]).wait()

      @pl.when(ch >= 2)
      def _():
        for p in range(_LPC * 4):
          pltpu.make_async_copy(ob.at[pl.ds(0, 8), pl.ds(0, 128)],
                                out.at[0, 0, 0], osem[h]).wait()

      l0 = ch * _LPC

      @plsc.parallel_loop(0, _LPC * (_BPW // 16), unroll=2)
      def _(g, rb=rb, ob=ob, l0=l0):
        lp = g // (_BPW // 16)
        b0 = (g % (_BPW // 16)) * 16
        lens16 = lens_v[pl.ds(b0, 16)]
        mv = jnp.where(l0 + lp < lens16, jnp.float32(1.0), jnp.float32(0.0))
        for jj in range(16):
          m = mv[jj]
          r = lp * _BPW + b0 + jj
          col = jnp.full((16,), b0 + jj, jnp.int32)
          plsc.store_scatter(ob, [rv0 + lp * 32, col],
                             rb[r, pl.ds(0, 16)] * m)
          plsc.store_scatter(ob, [rv1 + lp * 32, col],
                             rb[r, pl.ds(16, 16)] * m)

      for lp in range(_LPC):
        for cb in range(4):
          pltpu.async_copy(
              ob.at[pl.ds((lp * 4 + cb) * 8, 8), pl.ds(0, 128)],
              out.at[l0 + lp, cb, wid], osem[h])

      @pl.when(ch + 2 < _NLC)
      def _():
        build_idx(ch + 2, h)
        fire_gather(h)
    return 0

  lax.fori_loop(0, _NLC // 2, pair, 0)
  for h in (0, 1):
    for p in range(_LPC * 4):
      pltpu.make_async_copy(obs[h].at[pl.ds(0, 8), pl.ds(0, 128)],
                            out.at[0, 0, 0], osem[h]).wait()


_V = 1000000
_VPAD = 1000064
_DW = 32768
_NB = -(-_V // _DW)
_D1 = _NB * (_DW * 8 // 128)
_PLANE = _D1 * 128


def _dump_body(t_ref, o_ref):
  x = t_ref[...]
  x = x.reshape(8, _DW // 128, 128)
  x = jnp.transpose(x, (1, 0, 2))
  o_ref[...] = x.reshape(1, _DW * 8 // 128, 128)


def _tc_dump(table_t):
  return pl.pallas_call(
      _dump_body,
      grid=(4, _NB),
      in_specs=[pl.BlockSpec((8, _DW), lambda si, wb: (si, wb))],
      out_specs=pl.BlockSpec((1, _DW * 8 // 128, 128),
                             lambda si, wb: (si, wb, 0)),
      out_shape=jax.ShapeDtypeStruct((4, _D1, 128), jnp.float32),
  )(table_t)


_CW = 768
_CR = _CW // 128 * 8
_IP = 129
_NCH = 42


def _repack_body(td, t128, in0, in1, ob0, ob1, si0, si1, so0, so1):
  wid = lax.axis_index("s") * _NUM_CORES + lax.axis_index("c")
  start = wid * (_V // _NW) // 128 * 128
  end = jnp.where(wid == _NW - 1, _VPAD,
                  (wid + 1) * (_V // _NW) // 128 * 128)

  ins = (in0, in1)
  obs = (ob0, ob1)
  isem = (si0, si1)
  osem = (so0, so1)

  def cstart(k):
    return jnp.minimum(start + k * _CW, end - _CW)

  def fire_in(k, b):
    rs = cstart(k) // 128 * 8
    for si in range(4):
      pltpu.async_copy(td.at[si, pl.ds(rs, _CR), :],
                       ins[b].at[pl.ds(si * _CR, _CR), pl.ds(0, 128)],
                       isem[b])

  fire_in(0, 0)
  fire_in(1, 1)

  cj = lax.iota(jnp.int32, 16)
  crow0 = (cj // 8) * _CR + cj % 8
  crow1 = ((cj + 16) // 8) * _CR + cj % 8

  def pair(j, _):
    for h in (0, 1):
      k = 2 * j + h
      inb = ins[h]
      ob = obs[h]
      for si in range(4):
        pltpu.make_async_copy(td.at[0, pl.ds(0, _CR), :],
                              inb.at[pl.ds(si * _CR, _CR), pl.ds(0, 128)],
                              isem[h]).wait()

      @pl.when(k >= 2)
      def _():
        pltpu.make_async_copy(ob, t128.at[pl.ds(0, _CW * 32)],
                              osem[h]).wait()

      @plsc.parallel_loop(0, _CW, unroll=8)
      def _(w, inb=inb, ob=ob):
        wj8 = lax.shift_right_logical(w, 7) * 8
        u = jnp.full((16,), w & 127, jnp.int32)
        ob[pl.ds(w * 32, 16)] = plsc.load_gather(inb, [crow0 + wj8, u])
        ob[pl.ds(w * 32 + 16, 16)] = plsc.load_gather(inb, [crow1 + wj8, u])
      pltpu.async_copy(ob, t128.at[pl.ds(cstart(k) * 32, _CW * 32)], osem[h])

      @pl.when(k + 2 < _NCH)
      def _():
        fire_in(k + 2, h)
    return 0

  lax.fori_loop(0, _NCH // 2, pair, 0)
  pltpu.make_async_copy(ob0, t128.at[pl.ds(0, _CW * 32)], osem[0]).wait()
  pltpu.make_async_copy(ob1, t128.at[pl.ds(0, _CW * 32)], osem[1]).wait()


def _sc_repack(td):
  mesh = plsc.VectorSubcoreMesh(core_axis_name="c", subcore_axis_name="s",
                                num_cores=_NUM_CORES,
                                num_subcores=_NUM_SUBCORES)
  return pl.kernel(
      _repack_body,
      out_type=jax.ShapeDtypeStruct((_VPAD * _D,), jnp.float32),
      mesh=mesh,
      compiler_params=pltpu.CompilerParams(use_tc_tiling_on_sc=False,
                                           needs_layout_passes=False),
      scratch_types=[
          pltpu.VMEM((4 * _CR, _IP), jnp.float32),
          pltpu.VMEM((4 * _CR, _IP), jnp.float32),
          pltpu.VMEM((_CW * 32,), jnp.float32),
          pltpu.VMEM((_CW * 32,), jnp.float32),
          pltpu.SemaphoreType.DMA,
          pltpu.SemaphoreType.DMA,
          pltpu.SemaphoreType.DMA,
          pltpu.SemaphoreType.DMA,
      ],
  )(td)


def kernel(queries, query_lens, embedding_weight):
  q1d = queries.astype(jnp.int32).reshape(_ROWS)
  lens = query_lens.astype(jnp.int32)
  td = _tc_dump(embedding_weight.T)
  t128 = _sc_repack(td)
  table_rm = t128.reshape(_VPAD, _D)
  mesh = plsc.VectorSubcoreMesh(core_axis_name="c", subcore_axis_name="s",
                                num_cores=_NUM_CORES,
                                num_subcores=_NUM_SUBCORES)
  out5 = pl.kernel(
      _embed_body,
      out_type=jax.ShapeDtypeStruct((_L, 4, _NW, 8, 128), jnp.float32),
      mesh=mesh,
      compiler_params=pltpu.CompilerParams(use_tc_tiling_on_sc=False,
                                           needs_layout_passes=False),
      scratch_types=[
          pltpu.VMEM((_RPW,), jnp.int32),
          pltpu.VMEM((_BPW,), jnp.int32),
          pltpu.VMEM((_CR3,), jnp.int32),
          pltpu.VMEM((_CR3,), jnp.int32),
          pltpu.VMEM((_CR3, _D), jnp.float32),
          pltpu.VMEM((_CR3, _D), jnp.float32),
          pltpu.VMEM((_LPC * 4 * 8, _OPIT), jnp.float32),
          pltpu.VMEM((_LPC * 4 * 8, _OPIT), jnp.float32),
          pltpu.SemaphoreType.DMA,
          pltpu.SemaphoreType.DMA,
          pltpu.SemaphoreType.DMA,
          pltpu.SemaphoreType.DMA,
      ],
  )(table_rm, q1d, lens)
  return out5.transpose(2, 4, 0, 1, 3).reshape(_B, _L, _D)


# speedup vs baseline: 13.1067x; 1.1169x over previous
import jax
import jax.numpy as jnp
from jax import lax
from jax.experimental import pallas as pl
from jax.experimental.pallas import tpu as pltpu
from jax.experimental.pallas import tpu_sc as plsc

_NUM_CORES = 2
_NUM_SUBCORES = 16
_NW = _NUM_CORES * _NUM_SUBCORES

_B = 4096
_L = 50
_D = 32
_ROWS = _B * _L
_RPW = _ROWS // _NW
_BPW = _B // _NW
_SLEN = 128


_LPC = 5
_NLC = _L // _LPC
_CR3 = _LPC * _BPW
_OPIT = 129


def _embed_body(table, q1d, lens, out, idx_v, lens_v, ic0, ic1, rb0, rb1,
                ob0, ob1, g0, g1, o0, o1):
  wid = lax.axis_index("s") * _NUM_CORES + lax.axis_index("c")
  row0 = wid * _RPW

  pltpu.sync_copy(q1d.at[pl.ds(row0, _RPW)], idx_v)
  pltpu.sync_copy(lens.at[pl.ds(wid * _BPW, _BPW)], lens_v)

  ics = (ic0, ic1)
  rbs = (rb0, rb1)
  obs = (ob0, ob1)
  gsem = (g0, g1)
  osem = (o0, o1)

  b50 = lax.iota(jnp.int32, 16) * 50
  cj = lax.iota(jnp.int32, 16)
  rv0 = (cj // 8) * 8 + cj % 8
  rv1 = rv0 + 16

  def build_idx(ch, h):
    ic = ics[h]

    @plsc.parallel_loop(0, _LPC * (_BPW // 16), unroll=4)
    def _(g, ic=ic):
      lp = g // (_BPW // 16)
      b0 = (g % (_BPW // 16)) * 16
      v = plsc.load_gather(idx_v, [b50 + (b0 * 50 + ch * _LPC + lp)])
      ic[pl.ds(lp * _BPW + b0, 16)] = v

  def fire_gather(h):
    ic = ics[h]
    rb = rbs[h]
    for s in range(_CR3 // _SLEN):
      pltpu.async_copy(table.at[ic.at[pl.ds(s * _SLEN, _SLEN)]],
                       rb.at[pl.ds(s * _SLEN, _SLEN), :], gsem[h])

  build_idx(0, 0)
  fire_gather(0)
  build_idx(1, 1)
  fire_gather(1)

  def pair(j, _):
    for h in (0, 1):
      ch = 2 * j + h
      rb = rbs[h]
      ob = obs[h]
      for s in range(_CR3 // _SLEN):
        pltpu.make_async_copy(table.at[pl.ds(0, _SLEN)],
                              rb.at[pl.ds(0, _SLEN), :], gsem[h]).wait()

      @pl.when(ch >= 2)
      def _():
        for p in range(_LPC * 4):
          pltpu.make_async_copy(ob.at[pl.ds(0, 8), pl.ds(0, 128)],
                                out.at[0, 0, 0], osem[h]).wait()

      l0 = ch * _LPC

      @plsc.parallel_loop(0, _LPC * (_BPW // 16), unroll=2)
      def _(g, rb=rb, ob=ob, l0=l0):
        lp = g // (_BPW // 16)
        b0 = (g % (_BPW // 16)) * 16
        lens16 = lens_v[pl.ds(b0, 16)]
        mv = jnp.where(l0 + lp < lens16, jnp.float32(1.0), jnp.float32(0.0))
        for jj in range(16):
          m = mv[jj]
          r = lp * _BPW + b0 + jj
          col = jnp.full((16,), b0 + jj, jnp.int32)
          plsc.store_scatter(ob, [rv0 + lp * 32, col],
                             rb[r, pl.ds(0, 16)] * m)
          plsc.store_scatter(ob, [rv1 + lp * 32, col],
                             rb[r, pl.ds(16, 16)] * m)

      for lp in range(_LPC):
        for cb in range(4):
          pltpu.async_copy(
              ob.at[pl.ds((lp * 4 + cb) * 8, 8), pl.ds(0, 128)],
              out.at[l0 + lp, cb, wid], osem[h])

      @pl.when(ch + 2 < _NLC)
      def _():
        build_idx(ch + 2, h)
        fire_gather(h)
    return 0

  lax.fori_loop(0, _NLC // 2, pair, 0)
  for h in (0, 1):
    for p in range(_LPC * 4):
      pltpu.make_async_copy(obs[h].at[pl.ds(0, 8), pl.ds(0, 128)],
                            out.at[0, 0, 0], osem[h]).wait()


_V = 1000000
_VPAD = 1000064
_DW = 65536
_NB = -(-_V // _DW)
_D1 = _NB * (_DW * 8 // 128)
_PLANE = _D1 * 128


def _dump_body(t_ref, o_ref):
  x = t_ref[...]
  x = x.reshape(8, _DW // 128, 128)
  x = jnp.transpose(x, (1, 0, 2))
  o_ref[...] = x.reshape(1, _DW * 8 // 128, 128)


def _tc_dump(table_t):
  return pl.pallas_call(
      _dump_body,
      grid=(4, _NB),
      in_specs=[pl.BlockSpec((8, _DW), lambda si, wb: (si, wb))],
      out_specs=pl.BlockSpec((1, _DW * 8 // 128, 128),
                             lambda si, wb: (si, wb, 0)),
      out_shape=jax.ShapeDtypeStruct((4, _D1, 128), jnp.float32),
  )(table_t)


_CW = 768
_CR = _CW // 128 * 8
_IP = 129
_NCH = 42


def _repack_body(td, t128, in0, in1, ob0, ob1, si0, si1, so0, so1):
  wid = lax.axis_index("s") * _NUM_CORES + lax.axis_index("c")
  start = wid * (_V // _NW) // 128 * 128
  end = jnp.where(wid == _NW - 1, _VPAD,
                  (wid + 1) * (_V // _NW) // 128 * 128)

  ins = (in0, in1)
  obs = (ob0, ob1)
  isem = (si0, si1)
  osem = (so0, so1)

  def cstart(k):
    return jnp.minimum(start + k * _CW, end - _CW)

  def fire_in(k, b):
    rs = cstart(k) // 128 * 8
    for si in range(4):
      pltpu.async_copy(td.at[si, pl.ds(rs, _CR), :],
                       ins[b].at[pl.ds(si * _CR, _CR), pl.ds(0, 128)],
                       isem[b])

  fire_in(0, 0)
  fire_in(1, 1)

  cj = lax.iota(jnp.int32, 16)
  crow0 = (cj // 8) * _CR + cj % 8
  crow1 = ((cj + 16) // 8) * _CR + cj % 8

  def pair(j, _):
    for h in (0, 1):
      k = 2 * j + h
      inb = ins[h]
      ob = obs[h]
      for si in range(4):
        pltpu.make_async_copy(td.at[0, pl.ds(0, _CR), :],
                              inb.at[pl.ds(si * _CR, _CR), pl.ds(0, 128)],
                              isem[h]).wait()

      @pl.when(k >= 2)
      def _():
        pltpu.make_async_copy(ob, t128.at[pl.ds(0, _CW * 32)],
                              osem[h]).wait()

      @plsc.parallel_loop(0, _CW, unroll=8)
      def _(w, inb=inb, ob=ob):
        wj8 = lax.shift_right_logical(w, 7) * 8
        u = jnp.full((16,), w & 127, jnp.int32)
        ob[pl.ds(w * 32, 16)] = plsc.load_gather(inb, [crow0 + wj8, u])
        ob[pl.ds(w * 32 + 16, 16)] = plsc.load_gather(inb, [crow1 + wj8, u])
      pltpu.async_copy(ob, t128.at[pl.ds(cstart(k) * 32, _CW * 32)], osem[h])

      @pl.when(k + 2 < _NCH)
      def _():
        fire_in(k + 2, h)
    return 0

  lax.fori_loop(0, _NCH // 2, pair, 0)
  pltpu.make_async_copy(ob0, t128.at[pl.ds(0, _CW * 32)], osem[0]).wait()
  pltpu.make_async_copy(ob1, t128.at[pl.ds(0, _CW * 32)], osem[1]).wait()


def _sc_repack(td):
  mesh = plsc.VectorSubcoreMesh(core_axis_name="c", subcore_axis_name="s",
                                num_cores=_NUM_CORES,
                                num_subcores=_NUM_SUBCORES)
  return pl.kernel(
      _repack_body,
      out_type=jax.ShapeDtypeStruct((_VPAD * _D,), jnp.float32),
      mesh=mesh,
      compiler_params=pltpu.CompilerParams(use_tc_tiling_on_sc=False,
                                           needs_layout_passes=False),
      scratch_types=[
          pltpu.VMEM((4 * _CR, _IP), jnp.float32),
          pltpu.VMEM((4 * _CR, _IP), jnp.float32),
          pltpu.VMEM((_CW * 32,), jnp.float32),
          pltpu.VMEM((_CW * 32,), jnp.float32),
          pltpu.SemaphoreType.DMA,
          pltpu.SemaphoreType.DMA,
          pltpu.SemaphoreType.DMA,
          pltpu.SemaphoreType.DMA,
      ],
  )(td)


def kernel(queries, query_lens, embedding_weight):
  q1d = queries.astype(jnp.int32).reshape(_ROWS)
  lens = query_lens.astype(jnp.int32)
  td = _tc_dump(embedding_weight.T)
  t128 = _sc_repack(td)
  table_rm = t128.reshape(_VPAD, _D)
  mesh = plsc.VectorSubcoreMesh(core_axis_name="c", subcore_axis_name="s",
                                num_cores=_NUM_CORES,
                                num_subcores=_NUM_SUBCORES)
  out5 = pl.kernel(
      _embed_body,
      out_type=jax.ShapeDtypeStruct((_L, 4, _NW, 8, 128), jnp.float32),
      mesh=mesh,
      compiler_params=pltpu.CompilerParams(use_tc_tiling_on_sc=False,
                                           needs_layout_passes=False),
      scratch_types=[
          pltpu.VMEM((_RPW,), jnp.int32),
          pltpu.VMEM((_BPW,), jnp.int32),
          pltpu.VMEM((_CR3,), jnp.int32),
          pltpu.VMEM((_CR3,), jnp.int32),
          pltpu.VMEM((_CR3, _D), jnp.float32),
          pltpu.VMEM((_CR3, _D), jnp.float32),
          pltpu.VMEM((_LPC * 4 * 8, _OPIT), jnp.float32),
          pltpu.VMEM((_LPC * 4 * 8, _OPIT), jnp.float32),
          pltpu.SemaphoreType.DMA,
          pltpu.SemaphoreType.DMA,
          pltpu.SemaphoreType.DMA,
          pltpu.SemaphoreType.DMA,
      ],
  )(table_rm, q1d, lens)
  return out5.transpose(2, 4, 0, 1, 3).reshape(_B, _L, _D)


# speedup vs baseline: 13.9180x; 1.0619x over previous
import jax
import jax.numpy as jnp
from jax import lax
from jax.experimental import pallas as pl
from jax.experimental.pallas import tpu as pltpu
from jax.experimental.pallas import tpu_sc as plsc

_NUM_CORES = 2
_NUM_SUBCORES = 16
_NW = _NUM_CORES * _NUM_SUBCORES

_B = 4096
_L = 50
_D = 32
_ROWS = _B * _L
_RPW = _ROWS // _NW
_BPW = _B // _NW
_SLEN = 128


_LPC = 5
_NLC = _L // _LPC
_CR3 = _LPC * _BPW
_OPIT = 129


def _embed_body(table, q1d, lens, out, idx_v, lens_v, ic0, ic1, rb0, rb1,
                ob0, ob1, g0, g1, o0, o1):
  wid = lax.axis_index("s") * _NUM_CORES + lax.axis_index("c")
  row0 = wid * _RPW

  pltpu.sync_copy(q1d.at[pl.ds(row0, _RPW)], idx_v)
  pltpu.sync_copy(lens.at[pl.ds(wid * _BPW, _BPW)], lens_v)

  ics = (ic0, ic1)
  rbs = (rb0, rb1)
  obs = (ob0, ob1)
  gsem = (g0, g1)
  osem = (o0, o1)

  b50 = lax.iota(jnp.int32, 16) * 50
  cj = lax.iota(jnp.int32, 16)
  rv0 = (cj // 8) * 8 + cj % 8
  rv1 = rv0 + 16

  def build_idx(ch, h):
    ic = ics[h]

    @plsc.parallel_loop(0, _LPC * (_BPW // 16), unroll=4)
    def _(g, ic=ic):
      lp = g // (_BPW // 16)
      b0 = (g % (_BPW // 16)) * 16
      v = plsc.load_gather(idx_v, [b50 + (b0 * 50 + ch * _LPC + lp)])
      ic[pl.ds(lp * _BPW + b0, 16)] = v

  def fire_gather(h):
    ic = ics[h]
    rb = rbs[h]
    for s in range(_CR3 // _SLEN):
      pltpu.async_copy(table.at[ic.at[pl.ds(s * _SLEN, _SLEN)]],
                       rb.at[pl.ds(s * _SLEN, _SLEN), :], gsem[h])

  build_idx(0, 0)
  fire_gather(0)
  build_idx(1, 1)
  fire_gather(1)

  def pair(j, _):
    for h in (0, 1):
      ch = 2 * j + h
      rb = rbs[h]
      ob = obs[h]
      for s in range(_CR3 // _SLEN):
        pltpu.make_async_copy(table.at[pl.ds(0, _SLEN)],
                              rb.at[pl.ds(0, _SLEN), :], gsem[h]).wait()

      @pl.when(ch >= 2)
      def _():
        for p in range(_LPC * 4):
          pltpu.make_async_copy(ob.at[pl.ds(0, 8), pl.ds(0, 128)],
                                out.at[0, 0, 0], osem[h]).wait()

      l0 = ch * _LPC

      @plsc.parallel_loop(0, _LPC * (_BPW // 16), unroll=2)
      def _(g, rb=rb, ob=ob, l0=l0):
        lp = g // (_BPW // 16)
        b0 = (g % (_BPW // 16)) * 16
        lens16 = lens_v[pl.ds(b0, 16)]
        mv = jnp.where(l0 + lp < lens16, jnp.float32(1.0), jnp.float32(0.0))
        for jj in range(16):
          m = mv[jj]
          r = lp * _BPW + b0 + jj
          col = jnp.full((16,), b0 + jj, jnp.int32)
          plsc.store_scatter(ob, [rv0 + lp * 32, col],
                             rb[r, pl.ds(0, 16)] * m)
          plsc.store_scatter(ob, [rv1 + lp * 32, col],
                             rb[r, pl.ds(16, 16)] * m)

      for lp in range(_LPC):
        for cb in range(4):
          pltpu.async_copy(
              ob.at[pl.ds((lp * 4 + cb) * 8, 8), pl.ds(0, 128)],
              out.at[l0 + lp, cb, wid], osem[h])

      @pl.when(ch + 2 < _NLC)
      def _():
        build_idx(ch + 2, h)
        fire_gather(h)
    return 0

  lax.fori_loop(0, _NLC // 2, pair, 0)
  for h in (0, 1):
    for p in range(_LPC * 4):
      pltpu.make_async_copy(obs[h].at[pl.ds(0, 8), pl.ds(0, 128)],
                            out.at[0, 0, 0], osem[h]).wait()


_V = 1000000
_VPAD = 1000064
_DW = 131072
_NB = -(-_V // _DW)
_D1 = _NB * (_DW * 8 // 128)
_PLANE = _D1 * 128


def _dump_body(t_ref, o_ref):
  x = t_ref[...]
  x = x.reshape(8, _DW // 128, 128)
  x = jnp.transpose(x, (1, 0, 2))
  o_ref[...] = x.reshape(1, _DW * 8 // 128, 128)


def _tc_dump(table_t):
  return pl.pallas_call(
      _dump_body,
      grid=(4, _NB),
      in_specs=[pl.BlockSpec((8, _DW), lambda si, wb: (si, wb))],
      out_specs=pl.BlockSpec((1, _DW * 8 // 128, 128),
                             lambda si, wb: (si, wb, 0)),
      out_shape=jax.ShapeDtypeStruct((4, _D1, 128), jnp.float32),
  )(table_t)


_CW = 768
_CR = _CW // 128 * 8
_IP = 129
_NCH = 42


def _repack_body(td, t128, in0, in1, ob0, ob1, si0, si1, so0, so1):
  wid = lax.axis_index("s") * _NUM_CORES + lax.axis_index("c")
  start = wid * (_V // _NW) // 128 * 128
  end = jnp.where(wid == _NW - 1, _VPAD,
                  (wid + 1) * (_V // _NW) // 128 * 128)

  ins = (in0, in1)
  obs = (ob0, ob1)
  isem = (si0, si1)
  osem = (so0, so1)

  def cstart(k):
    return jnp.minimum(start + k * _CW, end - _CW)

  def fire_in(k, b):
    rs = cstart(k) // 128 * 8
    for si in range(4):
      pltpu.async_copy(td.at[si, pl.ds(rs, _CR), :],
                       ins[b].at[pl.ds(si * _CR, _CR), pl.ds(0, 128)],
                       isem[b])

  fire_in(0, 0)
  fire_in(1, 1)

  cj = lax.iota(jnp.int32, 16)
  crow0 = (cj // 8) * _CR + cj % 8
  crow1 = ((cj + 16) // 8) * _CR + cj % 8

  def pair(j, _):
    for h in (0, 1):
      k = 2 * j + h
      inb = ins[h]
      ob = obs[h]
      for si in range(4):
        pltpu.make_async_copy(td.at[0, pl.ds(0, _CR), :],
                              inb.at[pl.ds(si * _CR, _CR), pl.ds(0, 128)],
                              isem[h]).wait()

      @pl.when(k >= 2)
      def _():
        pltpu.make_async_copy(ob, t128.at[pl.ds(0, _CW * 32)],
                              osem[h]).wait()

      @plsc.parallel_loop(0, _CW, unroll=8)
      def _(w, inb=inb, ob=ob):
        wj8 = lax.shift_right_logical(w, 7) * 8
        u = jnp.full((16,), w & 127, jnp.int32)
        ob[pl.ds(w * 32, 16)] = plsc.load_gather(inb, [crow0 + wj8, u])
        ob[pl.ds(w * 32 + 16, 16)] = plsc.load_gather(inb, [crow1 + wj8, u])
      pltpu.async_copy(ob, t128.at[pl.ds(cstart(k) * 32, _CW * 32)], osem[h])

      @pl.when(k + 2 < _NCH)
      def _():
        fire_in(k + 2, h)
    return 0

  lax.fori_loop(0, _NCH // 2, pair, 0)
  pltpu.make_async_copy(ob0, t128.at[pl.ds(0, _CW * 32)], osem[0]).wait()
  pltpu.make_async_copy(ob1, t128.at[pl.ds(0, _CW * 32)], osem[1]).wait()


def _sc_repack(td):
  mesh = plsc.VectorSubcoreMesh(core_axis_name="c", subcore_axis_name="s",
                                num_cores=_NUM_CORES,
                                num_subcores=_NUM_SUBCORES)
  return pl.kernel(
      _repack_body,
      out_type=jax.ShapeDtypeStruct((_VPAD * _D,), jnp.float32),
      mesh=mesh,
      compiler_params=pltpu.CompilerParams(use_tc_tiling_on_sc=False,
                                           needs_layout_passes=False),
      scratch_types=[
          pltpu.VMEM((4 * _CR, _IP), jnp.float32),
          pltpu.VMEM((4 * _CR, _IP), jnp.float32),
          pltpu.VMEM((_CW * 32,), jnp.float32),
          pltpu.VMEM((_CW * 32,), jnp.float32),
          pltpu.SemaphoreType.DMA,
          pltpu.SemaphoreType.DMA,
          pltpu.SemaphoreType.DMA,
          pltpu.SemaphoreType.DMA,
      ],
  )(td)


def kernel(queries, query_lens, embedding_weight):
  q1d = queries.astype(jnp.int32).reshape(_ROWS)
  lens = query_lens.astype(jnp.int32)
  td = _tc_dump(embedding_weight.T)
  t128 = _sc_repack(td)
  table_rm = t128.reshape(_VPAD, _D)
  mesh = plsc.VectorSubcoreMesh(core_axis_name="c", subcore_axis_name="s",
                                num_cores=_NUM_CORES,
                                num_subcores=_NUM_SUBCORES)
  out5 = pl.kernel(
      _embed_body,
      out_type=jax.ShapeDtypeStruct((_L, 4, _NW, 8, 128), jnp.float32),
      mesh=mesh,
      compiler_params=pltpu.CompilerParams(use_tc_tiling_on_sc=False,
                                           needs_layout_passes=False),
      scratch_types=[
          pltpu.VMEM((_RPW,), jnp.int32),
          pltpu.VMEM((_BPW,), jnp.int32),
          pltpu.VMEM((_CR3,), jnp.int32),
          pltpu.VMEM((_CR3,), jnp.int32),
          pltpu.VMEM((_CR3, _D), jnp.float32),
          pltpu.VMEM((_CR3, _D), jnp.float32),
          pltpu.VMEM((_LPC * 4 * 8, _OPIT), jnp.float32),
          pltpu.VMEM((_LPC * 4 * 8, _OPIT), jnp.float32),
          pltpu.SemaphoreType.DMA,
          pltpu.SemaphoreType.DMA,
          pltpu.SemaphoreType.DMA,
          pltpu.SemaphoreType.DMA,
      ],
  )(table_rm, q1d, lens)
  return out5.transpose(2, 4, 0, 1, 3).reshape(_B, _L, _D)


# speedup vs baseline: 14.0633x; 1.0104x over previous
import jax
import jax.numpy as jnp
from jax import lax
from jax.experimental import pallas as pl
from jax.experimental.pallas import tpu as pltpu
from jax.experimental.pallas import tpu_sc as plsc

_NUM_CORES = 2
_NUM_SUBCORES = 16
_NW = _NUM_CORES * _NUM_SUBCORES

_B = 4096
_L = 50
_D = 32
_ROWS = _B * _L
_RPW = _ROWS // _NW
_BPW = _B // _NW
_SLEN = 128


_LPC = 5
_NLC = _L // _LPC
_CR3 = _LPC * _BPW
_OPIT = 129


def _embed_body(table, q1d, lens, out, idx_v, lens_v, ic0, ic1, rb0, rb1,
                ob0, ob1, g0, g1, o0, o1):
  wid = lax.axis_index("s") * _NUM_CORES + lax.axis_index("c")
  row0 = wid * _RPW

  pltpu.sync_copy(q1d.at[pl.ds(row0, _RPW)], idx_v)
  pltpu.sync_copy(lens.at[pl.ds(wid * _BPW, _BPW)], lens_v)

  ics = (ic0, ic1)
  rbs = (rb0, rb1)
  obs = (ob0, ob1)
  gsem = (g0, g1)
  osem = (o0, o1)

  b50 = lax.iota(jnp.int32, 16) * 50
  cj = lax.iota(jnp.int32, 16)
  rv0 = (cj // 8) * 8 + cj % 8
  rv1 = rv0 + 16

  def build_idx(ch, h):
    ic = ics[h]

    @plsc.parallel_loop(0, _LPC * (_BPW // 16), unroll=4)
    def _(g, ic=ic):
      lp = g // (_BPW // 16)
      b0 = (g % (_BPW // 16)) * 16
      v = plsc.load_gather(idx_v, [b50 + (b0 * 50 + ch * _LPC + lp)])
      ic[pl.ds(lp * _BPW + b0, 16)] = v

  def fire_gather(h):
    ic = ics[h]
    rb = rbs[h]
    for s in range(_CR3 // _SLEN):
      pltpu.async_copy(table.at[ic.at[pl.ds(s * _SLEN, _SLEN)]],
                       rb.at[pl.ds(s * _SLEN, _SLEN), :], gsem[h])

  build_idx(0, 0)
  fire_gather(0)
  build_idx(1, 1)
  fire_gather(1)

  def pair(j, _):
    for h in (0, 1):
      ch = 2 * j + h
      rb = rbs[h]
      ob = obs[h]
      for s in range(_CR3 // _SLEN):
        pltpu.make_async_copy(table.at[pl.ds(0, _SLEN)],
                              rb.at[pl.ds(0, _SLEN), :], gsem[h]).wait()

      @pl.when(ch >= 2)
      def _():
        for p in range(_LPC * 4):
          pltpu.make_async_copy(ob.at[pl.ds(0, 8), pl.ds(0, 128)],
                                out.at[0, 0, 0], osem[h]).wait()

      l0 = ch * _LPC

      @plsc.parallel_loop(0, _LPC * (_BPW // 16), unroll=2)
      def _(g, rb=rb, ob=ob, l0=l0):
        lp = g // (_BPW // 16)
        b0 = (g % (_BPW // 16)) * 16
        lens16 = lens_v[pl.ds(b0, 16)]
        mv = jnp.where(l0 + lp < lens16, jnp.float32(1.0), jnp.float32(0.0))
        for jj in range(16):
          m = mv[jj]
          r = lp * _BPW + b0 + jj
          col = jnp.full((16,), b0 + jj, jnp.int32)
          plsc.store_scatter(ob, [rv0 + lp * 32, col],
                             rb[r, pl.ds(0, 16)] * m)
          plsc.store_scatter(ob, [rv1 + lp * 32, col],
                             rb[r, pl.ds(16, 16)] * m)

      for lp in range(_LPC):
        for cb in range(4):
          pltpu.async_copy(
              ob.at[pl.ds((lp * 4 + cb) * 8, 8), pl.ds(0, 128)],
              out.at[l0 + lp, cb, wid], osem[h])

      @pl.when(ch + 2 < _NLC)
      def _():
        build_idx(ch + 2, h)
        fire_gather(h)
    return 0

  lax.fori_loop(0, _NLC // 2, pair, 0)
  for h in (0, 1):
    for p in range(_LPC * 4):
      pltpu.make_async_copy(obs[h].at[pl.ds(0, 8), pl.ds(0, 128)],
                            out.at[0, 0, 0], osem[h]).wait()


_V = 1000000
_VPAD = 1000064
_DW = 262144
_NB = -(-_V // _DW)
_D1 = _NB * (_DW * 8 // 128)
_PLANE = _D1 * 128


def _dump_body(t_ref, o_ref):
  x = t_ref[...]
  x = x.reshape(8, _DW // 128, 128)
  x = jnp.transpose(x, (1, 0, 2))
  o_ref[...] = x.reshape(1, _DW * 8 // 128, 128)


def _tc_dump(table_t):
  return pl.pallas_call(
      _dump_body,
      grid=(4, _NB),
      in_specs=[pl.BlockSpec((8, _DW), lambda si, wb: (si, wb))],
      out_specs=pl.BlockSpec((1, _DW * 8 // 128, 128),
                             lambda si, wb: (si, wb, 0)),
      out_shape=jax.ShapeDtypeStruct((4, _D1, 128), jnp.float32),
  )(table_t)


_CW = 768
_CR = _CW // 128 * 8
_IP = 129
_NCH = 42


def _repack_body(td, t128, in0, in1, ob0, ob1, si0, si1, so0, so1):
  wid = lax.axis_index("s") * _NUM_CORES + lax.axis_index("c")
  start = wid * (_V // _NW) // 128 * 128
  end = jnp.where(wid == _NW - 1, _VPAD,
                  (wid + 1) * (_V // _NW) // 128 * 128)

  ins = (in0, in1)
  obs = (ob0, ob1)
  isem = (si0, si1)
  osem = (so0, so1)

  def cstart(k):
    return jnp.minimum(start + k * _CW, end - _CW)

  def fire_in(k, b):
    rs = cstart(k) // 128 * 8
    for si in range(4):
      pltpu.async_copy(td.at[si, pl.ds(rs, _CR), :],
                       ins[b].at[pl.ds(si * _CR, _CR), pl.ds(0, 128)],
                       isem[b])

  fire_in(0, 0)
  fire_in(1, 1)

  cj = lax.iota(jnp.int32, 16)
  crow0 = (cj // 8) * _CR + cj % 8
  crow1 = ((cj + 16) // 8) * _CR + cj % 8

  def pair(j, _):
    for h in (0, 1):
      k = 2 * j + h
      inb = ins[h]
      ob = obs[h]
      for si in range(4):
        pltpu.make_async_copy(td.at[0, pl.ds(0, _CR), :],
                              inb.at[pl.ds(si * _CR, _CR), pl.ds(0, 128)],
                              isem[h]).wait()

      @pl.when(k >= 2)
      def _():
        pltpu.make_async_copy(ob, t128.at[pl.ds(0, _CW * 32)],
                              osem[h]).wait()

      @plsc.parallel_loop(0, _CW, unroll=8)
      def _(w, inb=inb, ob=ob):
        wj8 = lax.shift_right_logical(w, 7) * 8
        u = jnp.full((16,), w & 127, jnp.int32)
        ob[pl.ds(w * 32, 16)] = plsc.load_gather(inb, [crow0 + wj8, u])
        ob[pl.ds(w * 32 + 16, 16)] = plsc.load_gather(inb, [crow1 + wj8, u])
      pltpu.async_copy(ob, t128.at[pl.ds(cstart(k) * 32, _CW * 32)], osem[h])

      @pl.when(k + 2 < _NCH)
      def _():
        fire_in(k + 2, h)
    return 0

  lax.fori_loop(0, _NCH // 2, pair, 0)
  pltpu.make_async_copy(ob0, t128.at[pl.ds(0, _CW * 32)], osem[0]).wait()
  pltpu.make_async_copy(ob1, t128.at[pl.ds(0, _CW * 32)], osem[1]).wait()


def _sc_repack(td):
  mesh = plsc.VectorSubcoreMesh(core_axis_name="c", subcore_axis_name="s",
                                num_cores=_NUM_CORES,
                                num_subcores=_NUM_SUBCORES)
  return pl.kernel(
      _repack_body,
      out_type=jax.ShapeDtypeStruct((_VPAD * _D,), jnp.float32),
      mesh=mesh,
      compiler_params=pltpu.CompilerParams(use_tc_tiling_on_sc=False,
                                           needs_layout_passes=False),
      scratch_types=[
          pltpu.VMEM((4 * _CR, _IP), jnp.float32),
          pltpu.VMEM((4 * _CR, _IP), jnp.float32),
          pltpu.VMEM((_CW * 32,), jnp.float32),
          pltpu.VMEM((_CW * 32,), jnp.float32),
          pltpu.SemaphoreType.DMA,
          pltpu.SemaphoreType.DMA,
          pltpu.SemaphoreType.DMA,
          pltpu.SemaphoreType.DMA,
      ],
  )(td)


def kernel(queries, query_lens, embedding_weight):
  q1d = queries.astype(jnp.int32).reshape(_ROWS)
  lens = query_lens.astype(jnp.int32)
  td = _tc_dump(embedding_weight.T)
  t128 = _sc_repack(td)
  table_rm = t128.reshape(_VPAD, _D)
  mesh = plsc.VectorSubcoreMesh(core_axis_name="c", subcore_axis_name="s",
                                num_cores=_NUM_CORES,
                                num_subcores=_NUM_SUBCORES)
  out5 = pl.kernel(
      _embed_body,
      out_type=jax.ShapeDtypeStruct((_L, 4, _NW, 8, 128), jnp.float32),
      mesh=mesh,
      compiler_params=pltpu.CompilerParams(use_tc_tiling_on_sc=False,
                                           needs_layout_passes=False),
      scratch_types=[
          pltpu.VMEM((_RPW,), jnp.int32),
          pltpu.VMEM((_BPW,), jnp.int32),
          pltpu.VMEM((_CR3,), jnp.int32),
          pltpu.VMEM((_CR3,), jnp.int32),
          pltpu.VMEM((_CR3, _D), jnp.float32),
          pltpu.VMEM((_CR3, _D), jnp.float32),
          pltpu.VMEM((_LPC * 4 * 8, _OPIT), jnp.float32),
          pltpu.VMEM((_LPC * 4 * 8, _OPIT), jnp.float32),
          pltpu.SemaphoreType.DMA,
          pltpu.SemaphoreType.DMA,
          pltpu.SemaphoreType.DMA,
          pltpu.SemaphoreType.DMA,
      ],
  )(table_rm, q1d, lens)
  return out5.transpose(2, 4, 0, 1, 3).reshape(_B, _L, _D)
